# baseline (speedup 1.0000x reference)
import jax
import jax.numpy as jnp
from jax import lax
from jax.experimental import pallas as pl
from jax.experimental.pallas import tpu as pltpu
from jax.experimental.pallas import tpu_sc as plsc

N = 100000
D = 128
B = 1024
OUT = 128

NC = 2
NS = 16
NW = NC * NS
RA = 3128
RB = 3120
NSPLIT = 20
ROW_SPLIT = 16 * RA
CH = 128
NFULL = 24
BWIN = 160
BPAD = 32
BPS = B // NS
DUMMY = B


def _sc_body(x_hbm, bat_hbm, sums_hbm, cnth_hbm, maxh_hbm, emax_hbm,
             eid_hbm, ecnt_hbm, x0, b0, i0, x1, b1, i1, mflush, cflush,
             ebuf, eid_buf, ecnt_buf, zrow, spm_sums,
             sx0, sb0, ss0, sx1, sb1, ss1):
    c = lax.axis_index("c")
    s = lax.axis_index("s")
    wid = c * NS + s

    zv = jnp.zeros((16,), jnp.float32)

    def _zfill(r, _):
        for k in range(D // 16):
            zrow[r, pl.ds(k * 16, 16)] = zv
        return 0
    lax.fori_loop(0, BPS, _zfill, 0)

    rows0 = s * BPS
    pltpu.sync_copy(zrow, spm_sums.at[pl.ds(rows0, BPS), :])

    pltpu.sync_copy(bat_hbm.at[pl.ds(ROW_SPLIT + 16, 16)], b0.at[pl.ds(0, 16)])
    tsplit = b0[pl.ds(0, 16)][0]
    zlo = jnp.where(c == 0, jnp.int32(0), tsplit)
    zhi = jnp.where(c == 0, tsplit + 1, jnp.int32(B))

    def _zcnt(k, _):
        r = zlo + s + 16 * k

        @pl.when(r < zhi)
        def _():
            pltpu.sync_copy(zrow.at[0, pl.ds(0, 16)], cnth_hbm.at[r, 0])
        return 0
    lax.fori_loop(0, B // 16, _zcnt, 0)

    plsc.subcore_barrier()

    base = jnp.where(wid < NSPLIT, wid * RA,
                     NSPLIT * RA + (wid - NSPLIT) * RB)
    rows_w = jnp.where(wid < NSPLIT, jnp.int32(RA), jnp.int32(RB))
    minf = jnp.full((16,), -jnp.inf, jnp.float32)

    BUFS = ((x0, b0, i0, sx0, sb0, ss0),
            (x1, b1, i1, sx1, sb1, ss1))

    def issue_in(cb, p):
        xb, bb, ib, sx, sb, ss = BUFS[p]
        pltpu.async_copy(x_hbm.at[pl.ds(cb, CH), :], xb, sx)
        pltpu.async_copy(bat_hbm.at[pl.ds(cb, BWIN)], bb, sb)

    def wait_in(p):
        xb, bb, ib, sx, sb, ss = BUFS[p]
        pltpu.make_async_copy(x_hbm.at[pl.ds(0, CH), :], xb, sx).wait()
        pltpu.make_async_copy(bat_hbm.at[pl.ds(0, BWIN)], bb, sb).wait()

    def stage_idx(p):
        xb, bb, ib, sx, sb, ss = BUFS[p]
        for grp in range(CH // 16):
            ib[pl.ds(grp * 16, 16)] = bb[pl.ds(16 + grp * 16, 16)]

    def issue_scat(p):
        xb, bb, ib, sx, sb, ss = BUFS[p]
        pltpu.async_copy(xb, spm_sums.at[ib], ss, add=True)

    def wait_scat(p):
        xb, bb, ib, sx, sb, ss = BUFS[p]
        pltpu.make_async_copy(xb, spm_sums.at[ib], ss).wait()

    pltpu.sync_copy(bat_hbm.at[pl.ds(base + 16, 16)], b0.at[pl.ds(0, 16)])
    cur0 = b0[pl.ds(0, 16)][0]

    def flush_run(sid, runc_l, cnt_f, m):
        first = runc_l == 0

        @pl.when(first)
        def _():
            for k in range(D // 16):
                ebuf[pl.ds(k * 16, 16)] = m[k]
            eid_buf[pl.ds(0, 16)] = jnp.full((16,), sid, jnp.int32)
            ecnt_buf[pl.ds(0, 16)] = jnp.full((16,), cnt_f, jnp.float32)

        @pl.when(jnp.logical_not(first))
        def _():
            for k in range(D // 16):
                mflush[pl.ds(k * 16, 16)] = m[k]
            cflush[pl.ds(0, 16)] = jnp.full((16,), cnt_f, jnp.float32)
            pltpu.sync_copy(mflush, maxh_hbm.at[sid, 0])
            pltpu.sync_copy(cflush, cnth_hbm.at[sid, 0])

    def proc_rows(p, cb, carry):
        xb, bb, ib, sx, sb, ss = BUFS[p]

        def grp_body(gi, gc):
            cur = gc[0]
            runc_l = gc[1]
            rst = gc[2]
            m = list(gc[3:])
            gstart = gi * 16
            for r in range(16):
                sv = bb[pl.ds(16 + gstart + r, 16)][0]
                ch = sv != cur
                jg = cb + gstart + r

                @pl.when(ch)
                def _(sid=cur, rl=runc_l, cf=(jg - rst).astype(jnp.float32),
                      mm=tuple(m)):
                    flush_run(sid, rl, cf, mm)

                newm = []
                for k in range(D // 16):
                    xk = xb[gstart + r, pl.ds(k * 16, 16)]
                    newm.append(jnp.where(ch, xk, jnp.maximum(m[k], xk)))
                m = newm
                runc_l = runc_l + ch.astype(jnp.int32)
                rst = jnp.where(ch, jg, rst)
                cur = sv
            return (cur, runc_l, rst) + tuple(m)

        return lax.fori_loop(0, CH // 16, grp_body, carry)

    issue_in(base, 0)
    carry = (cur0, jnp.int32(0), base) + tuple(minf for _ in range(D // 16))
    wait_in(0)
    stage_idx(0)
    issue_scat(0)
    issue_in(base + CH, 1)
    carry = proc_rows(0, base, carry)

    def pair_body(q, carry):
        cb = base + (2 * q + 1) * CH
        wait_in(1)
        stage_idx(1)
        issue_scat(1)
        wait_scat(0)
        issue_in(cb + CH, 0)
        carry = proc_rows(1, cb, carry)

        wait_in(0)
        stage_idx(0)
        issue_scat(0)
        wait_scat(1)
        issue_in(cb + 2 * CH, 1)
        carry = proc_rows(0, cb + CH, carry)
        return carry

    carry = lax.fori_loop(0, (NFULL - 2) // 2, pair_body, carry)

    cbl = base + rows_w - CH
    wait_in(1)
    stage_idx(1)
    issue_scat(1)
    wait_scat(0)
    issue_in(cbl, 0)
    carry = proc_rows(1, base + (NFULL - 1) * CH, carry)

    lanes = lax.iota(jnp.int32, 16)
    ovl = (NFULL + 1) * CH - rows_w
    wait_in(0)
    for grp in range(CH // 16):
        bvs = b0[pl.ds(16 + grp * 16, 16)]
        pos = lanes + grp * 16
        i0[pl.ds(grp * 16, 16)] = jnp.where(pos < ovl, jnp.int32(DUMMY), bvs)
    issue_scat(0)
    wait_scat(1)

    def row_body(j, rc):
        cur = rc[0]
        runc = rc[1]
        rst = rc[2]
        m = rc[3:]
        sv = b0[pl.ds(16 + j, 16)][0]
        changed = sv != cur
        jg = cbl + j

        @pl.when(changed)
        def _():
            flush_run(cur, runc, (jg - rst).astype(jnp.float32), m)

        newm = []
        for k in range(D // 16):
            xk = x0[j, pl.ds(k * 16, 16)]
            newm.append(jnp.where(changed, xk, jnp.maximum(m[k], xk)))
        return (sv, runc + changed.astype(jnp.int32),
                jnp.where(changed, jg, rst)) + tuple(newm)

    carry = lax.fori_loop(ovl, CH, row_body, carry)
    wait_scat(0)

    cur = carry[0]
    runc = carry[1]
    rst = carry[2]
    m = carry[3:]
    endcnt = (base + rows_w - rst).astype(jnp.float32)
    for k in range(D // 16):
        ebuf[pl.ds(D + k * 16, 16)] = m[k]
    eid_buf[pl.ds(16, 16)] = jnp.full((16,), cur, jnp.int32)
    ecnt_buf[pl.ds(16, 16)] = jnp.full((16,), endcnt, jnp.float32)

    @pl.when(runc == 0)
    def _single_run():
        for k in range(D // 16):
            ebuf[pl.ds(k * 16, 16)] = m[k]
        eid_buf[pl.ds(0, 16)] = jnp.full((16,), cur, jnp.int32)
        ecnt_buf[pl.ds(0, 16)] = jnp.zeros((16,), jnp.float32)

    pltpu.sync_copy(ebuf, emax_hbm.at[wid])
    pltpu.sync_copy(eid_buf, eid_hbm.at[wid])
    pltpu.sync_copy(ecnt_buf, ecnt_hbm.at[wid])

    plsc.subcore_barrier()
    pltpu.sync_copy(spm_sums.at[pl.ds(rows0, BPS), :],
                    sums_hbm.at[c, pl.ds(rows0, BPS), :])


def _tc_body(sums2, cnth, maxh, emax, eid, ecnt, w_ref, b_ref,
             z_ref, out_ref, mx, ct):
    sums = sums2[0] + sums2[1]

    ct[...] = cnth[...]
    z16 = jnp.zeros((1, 16), jnp.float32)

    def _cclear(i, _):
        sid = eid[i, 0]
        ct[pl.ds(sid, 1), :] = z16
        return 0
    lax.fori_loop(0, 2 * NW, _cclear, 0)

    def _cadd(i, _):
        sid = eid[i, 0]
        ct[pl.ds(sid, 1), :] = ct[pl.ds(sid, 1), :] + ecnt[pl.ds(i, 1), :]
        return 0
    lax.fori_loop(0, 2 * NW, _cadd, 0)

    cnt = ct[:, 0:1]
    mx[...] = jnp.where(cnt > 0.0, maxh[...], -jnp.inf)

    neg = jnp.full((1, D), -jnp.inf, jnp.float32)

    def _clear(i, _):
        sid = eid[i, 0]
        mx[pl.ds(sid, 1), :] = neg
        return 0
    lax.fori_loop(0, 2 * NW, _clear, 0)

    def _apply(i, _):
        sid = eid[i, 0]
        row = emax[pl.ds(i, 1), :]
        mx[pl.ds(sid, 1), :] = jnp.maximum(mx[pl.ds(sid, 1), :], row)
        return 0
    lax.fori_loop(0, 2 * NW, _apply, 0)

    mxv = mx[...]
    mxv = jnp.where(jnp.isfinite(mxv), mxv, 0.0)
    mean = sums / jnp.maximum(cnt, 1.0)
    z = jnp.concatenate([mean, mxv, sums], axis=1)
    z_ref[...] = z
    out_ref[...] = jnp.dot(z, w_ref[...],
                           preferred_element_type=jnp.float32) + b_ref[...]


def kernel(x, batch, W, b):
    batch_pad = jnp.concatenate([jnp.zeros((16,), jnp.int32), batch,
                                 jnp.zeros((BPAD,), jnp.int32)])

    mesh = plsc.VectorSubcoreMesh(core_axis_name="c", subcore_axis_name="s",
                                  num_cores=NC, num_subcores=NS)
    sc = pl.kernel(
        _sc_body,
        out_type=(
            jax.ShapeDtypeStruct((NC, B, D), jnp.float32),
            jax.ShapeDtypeStruct((B, 1, 16), jnp.float32),
            jax.ShapeDtypeStruct((B, 1, D), jnp.float32),
            jax.ShapeDtypeStruct((NW, 2 * D), jnp.float32),
            jax.ShapeDtypeStruct((NW, 32), jnp.int32),
            jax.ShapeDtypeStruct((NW, 32), jnp.float32),
        ),
        mesh=mesh,
        scratch_types=[
            pltpu.VMEM((CH, D), jnp.float32),
            pltpu.VMEM((BWIN,), jnp.int32),
            pltpu.VMEM((CH,), jnp.int32),
            pltpu.VMEM((CH, D), jnp.float32),
            pltpu.VMEM((BWIN,), jnp.int32),
            pltpu.VMEM((CH,), jnp.int32),
            pltpu.VMEM((D,), jnp.float32),
            pltpu.VMEM((16,), jnp.float32),
            pltpu.VMEM((2 * D,), jnp.float32),
            pltpu.VMEM((32,), jnp.int32),
            pltpu.VMEM((32,), jnp.float32),
            pltpu.VMEM((BPS, D), jnp.float32),
            pltpu.VMEM_SHARED((B + 8, D), jnp.float32),
            pltpu.SemaphoreType.DMA,
            pltpu.SemaphoreType.DMA,
            pltpu.SemaphoreType.DMA,
            pltpu.SemaphoreType.DMA,
            pltpu.SemaphoreType.DMA,
            pltpu.SemaphoreType.DMA,
        ],
    )
    sums2, cnth, maxh, emax, eid, ecnt = sc(x, batch_pad)

    z, logits = pl.pallas_call(
        _tc_body,
        out_shape=[
            jax.ShapeDtypeStruct((B, 3 * D), jnp.float32),
            jax.ShapeDtypeStruct((B, OUT), jnp.float32),
        ],
        in_specs=[
            pl.BlockSpec(memory_space=pltpu.VMEM),
            pl.BlockSpec(memory_space=pltpu.VMEM),
            pl.BlockSpec(memory_space=pltpu.VMEM),
            pl.BlockSpec(memory_space=pltpu.VMEM),
            pl.BlockSpec(memory_space=pltpu.SMEM),
            pl.BlockSpec(memory_space=pltpu.VMEM),
            pl.BlockSpec(memory_space=pltpu.VMEM),
            pl.BlockSpec(memory_space=pltpu.VMEM),
        ],
        scratch_shapes=[pltpu.VMEM((B, D), jnp.float32),
                        pltpu.VMEM((B, 16), jnp.float32)],
    )(sums2, cnth.reshape(B, 16), maxh.reshape(B, D), emax.reshape(2 * NW, D),
      eid.reshape(2 * NW, 16), ecnt.reshape(2 * NW, 16), W, b.reshape(1, OUT))
    return (z, logits)


# speedup vs baseline: 1.0112x; 1.0112x over previous
import jax
import jax.numpy as jnp
from jax import lax
from jax.experimental import pallas as pl
from jax.experimental.pallas import tpu as pltpu
from jax.experimental.pallas import tpu_sc as plsc

N = 100000
D = 128
B = 1024
OUT = 128

NC = 2
NS = 16
NW = NC * NS
RA = 3128
RB = 3120
NSPLIT = 20
ROW_SPLIT = 16 * RA
CH = 128
NFULL = 24
BWIN = 160
BPAD = 32
BPS = B // NS
DUMMY = B


def _sc_body(x_hbm, bat_hbm, sums_hbm, cnth_hbm, maxh_hbm, emax_hbm,
             eid_hbm, ecnt_hbm, x0, b0, i0, x1, b1, i1, mflush, cflush,
             ebuf, eid_buf, ecnt_buf, zrow, spm_sums,
             sx0, sb0, ss0, sx1, sb1, ss1, szc):
    c = lax.axis_index("c")
    s = lax.axis_index("s")
    wid = c * NS + s

    zv = jnp.zeros((16,), jnp.float32)

    def _zfill(r, _):
        for k in range(D // 16):
            zrow[r, pl.ds(k * 16, 16)] = zv
        return 0
    lax.fori_loop(0, BPS, _zfill, 0)

    rows0 = s * BPS
    pltpu.sync_copy(zrow, spm_sums.at[pl.ds(rows0, BPS), :])

    pltpu.sync_copy(bat_hbm.at[pl.ds(ROW_SPLIT + 16, 16)], b0.at[pl.ds(0, 16)])
    tsplit = b0[pl.ds(0, 16)][0]
    zlo = jnp.where(c == 0, jnp.int32(0), tsplit)
    zhi = jnp.where(c == 0, tsplit + 1, jnp.int32(B))

    def _zcnt_issue(k, _):
        r = zlo + s + 16 * k

        @pl.when(r < zhi)
        def _():
            pltpu.async_copy(zrow.at[0, pl.ds(0, 16)], cnth_hbm.at[r, 0], szc)
        return 0
    lax.fori_loop(0, B // 16, _zcnt_issue, 0)

    def _zcnt_drain(k, _):
        r = zlo + s + 16 * k

        @pl.when(r < zhi)
        def _():
            pltpu.make_async_copy(zrow.at[0, pl.ds(0, 16)],
                                  cnth_hbm.at[r, 0], szc).wait()
        return 0
    lax.fori_loop(0, B // 16, _zcnt_drain, 0)

    plsc.subcore_barrier()

    base = jnp.where(wid < NSPLIT, wid * RA,
                     NSPLIT * RA + (wid - NSPLIT) * RB)
    rows_w = jnp.where(wid < NSPLIT, jnp.int32(RA), jnp.int32(RB))
    minf = jnp.full((16,), -jnp.inf, jnp.float32)

    BUFS = ((x0, b0, i0, sx0, sb0, ss0),
            (x1, b1, i1, sx1, sb1, ss1))

    def issue_in(cb, p):
        xb, bb, ib, sx, sb, ss = BUFS[p]
        pltpu.async_copy(x_hbm.at[pl.ds(cb, CH), :], xb, sx)
        pltpu.async_copy(bat_hbm.at[pl.ds(cb, BWIN)], bb, sb)

    def wait_in(p):
        xb, bb, ib, sx, sb, ss = BUFS[p]
        pltpu.make_async_copy(x_hbm.at[pl.ds(0, CH), :], xb, sx).wait()
        pltpu.make_async_copy(bat_hbm.at[pl.ds(0, BWIN)], bb, sb).wait()

    def stage_idx(p):
        xb, bb, ib, sx, sb, ss = BUFS[p]
        for grp in range(CH // 16):
            ib[pl.ds(grp * 16, 16)] = bb[pl.ds(16 + grp * 16, 16)]

    def issue_scat(p):
        xb, bb, ib, sx, sb, ss = BUFS[p]
        pltpu.async_copy(xb, spm_sums.at[ib], ss, add=True)

    def wait_scat(p):
        xb, bb, ib, sx, sb, ss = BUFS[p]
        pltpu.make_async_copy(xb, spm_sums.at[ib], ss).wait()

    pltpu.sync_copy(bat_hbm.at[pl.ds(base + 16, 16)], b0.at[pl.ds(0, 16)])
    cur0 = b0[pl.ds(0, 16)][0]

    def flush_run(sid, runc_l, cnt_f, m):
        first = runc_l == 0

        @pl.when(first)
        def _():
            for k in range(D // 16):
                ebuf[pl.ds(k * 16, 16)] = m[k]
            eid_buf[pl.ds(0, 16)] = jnp.full((16,), sid, jnp.int32)
            ecnt_buf[pl.ds(0, 16)] = jnp.full((16,), cnt_f, jnp.float32)

        @pl.when(jnp.logical_not(first))
        def _():
            for k in range(D // 16):
                mflush[pl.ds(k * 16, 16)] = m[k]
            cflush[pl.ds(0, 16)] = jnp.full((16,), cnt_f, jnp.float32)
            pltpu.sync_copy(mflush, maxh_hbm.at[sid, 0])
            pltpu.sync_copy(cflush, cnth_hbm.at[sid, 0])

    def proc_rows(p, cb, carry):
        xb, bb, ib, sx, sb, ss = BUFS[p]

        def grp_body(gi, gc):
            cur = gc[0]
            runc_l = gc[1]
            rst = gc[2]
            m = list(gc[3:])
            gstart = gi * 16
            for r in range(16):
                sv = bb[pl.ds(16 + gstart + r, 16)][0]
                ch = sv != cur
                jg = cb + gstart + r

                @pl.when(ch)
                def _(sid=cur, rl=runc_l, cf=(jg - rst).astype(jnp.float32),
                      mm=tuple(m)):
                    flush_run(sid, rl, cf, mm)

                newm = []
                for k in range(D // 16):
                    xk = xb[gstart + r, pl.ds(k * 16, 16)]
                    newm.append(jnp.where(ch, xk, jnp.maximum(m[k], xk)))
                m = newm
                runc_l = runc_l + ch.astype(jnp.int32)
                rst = jnp.where(ch, jg, rst)
                cur = sv
            return (cur, runc_l, rst) + tuple(m)

        return lax.fori_loop(0, CH // 16, grp_body, carry)

    issue_in(base, 0)
    carry = (cur0, jnp.int32(0), base) + tuple(minf for _ in range(D // 16))
    wait_in(0)
    stage_idx(0)
    issue_scat(0)
    issue_in(base + CH, 1)
    carry = proc_rows(0, base, carry)

    def pair_body(q, carry):
        cb = base + (2 * q + 1) * CH
        wait_in(1)
        stage_idx(1)
        issue_scat(1)
        wait_scat(0)
        issue_in(cb + CH, 0)
        carry = proc_rows(1, cb, carry)

        wait_in(0)
        stage_idx(0)
        issue_scat(0)
        wait_scat(1)
        issue_in(cb + 2 * CH, 1)
        carry = proc_rows(0, cb + CH, carry)
        return carry

    carry = lax.fori_loop(0, (NFULL - 2) // 2, pair_body, carry)

    cbl = base + rows_w - CH
    wait_in(1)
    stage_idx(1)
    issue_scat(1)
    wait_scat(0)
    issue_in(cbl, 0)
    carry = proc_rows(1, base + (NFULL - 1) * CH, carry)

    lanes = lax.iota(jnp.int32, 16)
    ovl = (NFULL + 1) * CH - rows_w
    wait_in(0)
    for grp in range(CH // 16):
        bvs = b0[pl.ds(16 + grp * 16, 16)]
        pos = lanes + grp * 16
        i0[pl.ds(grp * 16, 16)] = jnp.where(pos < ovl, jnp.int32(DUMMY), bvs)
    issue_scat(0)
    wait_scat(1)

    def row_body(j, rc):
        cur = rc[0]
        runc = rc[1]
        rst = rc[2]
        m = rc[3:]
        sv = b0[pl.ds(16 + j, 16)][0]
        changed = sv != cur
        jg = cbl + j

        @pl.when(changed)
        def _():
            flush_run(cur, runc, (jg - rst).astype(jnp.float32), m)

        newm = []
        for k in range(D // 16):
            xk = x0[j, pl.ds(k * 16, 16)]
            newm.append(jnp.where(changed, xk, jnp.maximum(m[k], xk)))
        return (sv, runc + changed.astype(jnp.int32),
                jnp.where(changed, jg, rst)) + tuple(newm)

    carry = lax.fori_loop(ovl, CH, row_body, carry)
    wait_scat(0)

    cur = carry[0]
    runc = carry[1]
    rst = carry[2]
    m = carry[3:]
    endcnt = (base + rows_w - rst).astype(jnp.float32)
    for k in range(D // 16):
        ebuf[pl.ds(D + k * 16, 16)] = m[k]
    eid_buf[pl.ds(16, 16)] = jnp.full((16,), cur, jnp.int32)
    ecnt_buf[pl.ds(16, 16)] = jnp.full((16,), endcnt, jnp.float32)

    @pl.when(runc == 0)
    def _single_run():
        for k in range(D // 16):
            ebuf[pl.ds(k * 16, 16)] = m[k]
        eid_buf[pl.ds(0, 16)] = jnp.full((16,), cur, jnp.int32)
        ecnt_buf[pl.ds(0, 16)] = jnp.zeros((16,), jnp.float32)

    pltpu.sync_copy(ebuf, emax_hbm.at[wid])
    pltpu.sync_copy(eid_buf, eid_hbm.at[wid])
    pltpu.sync_copy(ecnt_buf, ecnt_hbm.at[wid])

    plsc.subcore_barrier()
    pltpu.sync_copy(spm_sums.at[pl.ds(rows0, BPS), :],
                    sums_hbm.at[c, pl.ds(rows0, BPS), :])


def _tc_body(sums2, cnth, maxh, emax, eid, ecnt, w_ref, b_ref,
             z_ref, out_ref, mx, ct):
    sums = sums2[0] + sums2[1]

    ct[...] = cnth[...]
    z16 = jnp.zeros((1, 16), jnp.float32)

    def _cclear(i, _):
        sid = eid[i, 0]
        ct[pl.ds(sid, 1), :] = z16
        return 0
    lax.fori_loop(0, 2 * NW, _cclear, 0)

    def _cadd(i, _):
        sid = eid[i, 0]
        ct[pl.ds(sid, 1), :] = ct[pl.ds(sid, 1), :] + ecnt[pl.ds(i, 1), :]
        return 0
    lax.fori_loop(0, 2 * NW, _cadd, 0)

    cnt = ct[:, 0:1]
    mx[...] = jnp.where(cnt > 0.0, maxh[...], -jnp.inf)

    neg = jnp.full((1, D), -jnp.inf, jnp.float32)

    def _clear(i, _):
        sid = eid[i, 0]
        mx[pl.ds(sid, 1), :] = neg
        return 0
    lax.fori_loop(0, 2 * NW, _clear, 0)

    def _apply(i, _):
        sid = eid[i, 0]
        row = emax[pl.ds(i, 1), :]
        mx[pl.ds(sid, 1), :] = jnp.maximum(mx[pl.ds(sid, 1), :], row)
        return 0
    lax.fori_loop(0, 2 * NW, _apply, 0)

    mxv = mx[...]
    mxv = jnp.where(jnp.isfinite(mxv), mxv, 0.0)
    mean = sums / jnp.maximum(cnt, 1.0)
    z = jnp.concatenate([mean, mxv, sums], axis=1)
    z_ref[...] = z
    out_ref[...] = jnp.dot(z, w_ref[...],
                           preferred_element_type=jnp.float32) + b_ref[...]


def kernel(x, batch, W, b):
    batch_pad = jnp.concatenate([jnp.zeros((16,), jnp.int32), batch,
                                 jnp.zeros((BPAD,), jnp.int32)])

    mesh = plsc.VectorSubcoreMesh(core_axis_name="c", subcore_axis_name="s",
                                  num_cores=NC, num_subcores=NS)
    sc = pl.kernel(
        _sc_body,
        out_type=(
            jax.ShapeDtypeStruct((NC, B, D), jnp.float32),
            jax.ShapeDtypeStruct((B, 1, 16), jnp.float32),
            jax.ShapeDtypeStruct((B, 1, D), jnp.float32),
            jax.ShapeDtypeStruct((NW, 2 * D), jnp.float32),
            jax.ShapeDtypeStruct((NW, 32), jnp.int32),
            jax.ShapeDtypeStruct((NW, 32), jnp.float32),
        ),
        mesh=mesh,
        scratch_types=[
            pltpu.VMEM((CH, D), jnp.float32),
            pltpu.VMEM((BWIN,), jnp.int32),
            pltpu.VMEM((CH,), jnp.int32),
            pltpu.VMEM((CH, D), jnp.float32),
            pltpu.VMEM((BWIN,), jnp.int32),
            pltpu.VMEM((CH,), jnp.int32),
            pltpu.VMEM((D,), jnp.float32),
            pltpu.VMEM((16,), jnp.float32),
            pltpu.VMEM((2 * D,), jnp.float32),
            pltpu.VMEM((32,), jnp.int32),
            pltpu.VMEM((32,), jnp.float32),
            pltpu.VMEM((BPS, D), jnp.float32),
            pltpu.VMEM_SHARED((B + 8, D), jnp.float32),
            pltpu.SemaphoreType.DMA,
            pltpu.SemaphoreType.DMA,
            pltpu.SemaphoreType.DMA,
            pltpu.SemaphoreType.DMA,
            pltpu.SemaphoreType.DMA,
            pltpu.SemaphoreType.DMA,
            pltpu.SemaphoreType.DMA,
        ],
    )
    sums2, cnth, maxh, emax, eid, ecnt = sc(x, batch_pad)

    z, logits = pl.pallas_call(
        _tc_body,
        out_shape=[
            jax.ShapeDtypeStruct((B, 3 * D), jnp.float32),
            jax.ShapeDtypeStruct((B, OUT), jnp.float32),
        ],
        in_specs=[
            pl.BlockSpec(memory_space=pltpu.VMEM),
            pl.BlockSpec(memory_space=pltpu.VMEM),
            pl.BlockSpec(memory_space=pltpu.VMEM),
            pl.BlockSpec(memory_space=pltpu.VMEM),
            pl.BlockSpec(memory_space=pltpu.SMEM),
            pl.BlockSpec(memory_space=pltpu.VMEM),
            pl.BlockSpec(memory_space=pltpu.VMEM),
            pl.BlockSpec(memory_space=pltpu.VMEM),
        ],
        scratch_shapes=[pltpu.VMEM((B, D), jnp.float32),
                        pltpu.VMEM((B, 16), jnp.float32)],
    )(sums2, cnth.reshape(B, 16), maxh.reshape(B, D), emax.reshape(2 * NW, D),
      eid.reshape(2 * NW, 16), ecnt.reshape(2 * NW, 16), W, b.reshape(1, OUT))
    return (z, logits)


# speedup vs baseline: 1.0511x; 1.0395x over previous
import jax
import jax.numpy as jnp
from jax import lax
from jax.experimental import pallas as pl
from jax.experimental.pallas import tpu as pltpu
from jax.experimental.pallas import tpu_sc as plsc

N = 100000
D = 128
B = 1024
OUT = 128

NC = 2
NS = 16
NW = NC * NS
RA = 3128
RB = 3120
NSPLIT = 20
CH = 128
CHUNK = 256
NFULL = 12
BWIN = 288
BPAD = 32
CNT_W = 128
BPS = B // NS
DUMMY = B


def _sc_body(x_hbm, bat_hbm, ones_hbm, sums_hbm, cnts_hbm, maxh_hbm, emax_hbm,
             eid_hbm, x0, b0, i0a, i0b, x1, b1, i1a, i1b, ones_buf, mflush, ebuf, eid_buf,
             zrow, zcnt, spm_sums, spm_cnts,
             sx0, sb0, ss0, sc0, sx1, sb1, ss1, sc1):
    c = lax.axis_index("c")
    s = lax.axis_index("s")
    wid = c * NS + s

    zv = jnp.zeros((16,), jnp.float32)
    lanes = lax.iota(jnp.int32, 16)

    def _zfill(r, _):
        for k in range(D // 16):
            zrow[r, pl.ds(k * 16, 16)] = zv
            zcnt[r, pl.ds(k * 16, 16)] = zv
        return 0
    lax.fori_loop(0, BPS, _zfill, 0)

    pltpu.sync_copy(ones_hbm, ones_buf)

    rows0 = s * BPS
    pltpu.sync_copy(zrow, spm_sums.at[pl.ds(rows0, BPS), :])
    pltpu.sync_copy(zcnt, spm_cnts.at[pl.ds(rows0, BPS), :])
    plsc.subcore_barrier()

    base = jnp.where(wid < NSPLIT, wid * RA,
                     NSPLIT * RA + (wid - NSPLIT) * RB)
    rows_w = jnp.where(wid < NSPLIT, jnp.int32(RA), jnp.int32(RB))
    minf = jnp.full((16,), -jnp.inf, jnp.float32)

    BUFS = ((x0, b0, i0a, i0b, sx0, sb0, ss0, sc0),
            (x1, b1, i1a, i1b, sx1, sb1, ss1, sc1))

    def issue_in(cb, p):
        xb, bb, ia, ib2, sx, sb, ss, scn = BUFS[p]
        pltpu.async_copy(x_hbm.at[pl.ds(cb, CHUNK), :], xb, sx)
        pltpu.async_copy(bat_hbm.at[pl.ds(cb, BWIN)], bb, sb)

    def wait_in(p):
        xb, bb, ia, ib2, sx, sb, ss, scn = BUFS[p]
        pltpu.make_async_copy(x_hbm.at[pl.ds(0, CHUNK), :], xb, sx).wait()
        pltpu.make_async_copy(bat_hbm.at[pl.ds(0, BWIN)], bb, sb).wait()

    def stage_idx(p):
        xb, bb, ia, ib2, sx, sb, ss, scn = BUFS[p]
        for grp in range(CH // 16):
            ia[pl.ds(grp * 16, 16)] = bb[pl.ds(16 + grp * 16, 16)]
            ib2[pl.ds(grp * 16, 16)] = bb[pl.ds(16 + CH + grp * 16, 16)]

    def issue_scat(p):
        xb, bb, ia, ib2, sx, sb, ss, scn = BUFS[p]
        pltpu.async_copy(xb.at[pl.ds(0, CH), :], spm_sums.at[ia], ss, add=True)
        pltpu.async_copy(xb.at[pl.ds(CH, CH), :], spm_sums.at[ib2], ss,
                         add=True)
        pltpu.async_copy(ones_buf, spm_cnts.at[ia], scn, add=True)
        pltpu.async_copy(ones_buf, spm_cnts.at[ib2], scn, add=True)

    def wait_scat(p):
        xb, bb, ia, ib2, sx, sb, ss, scn = BUFS[p]
        pltpu.make_async_copy(xb.at[pl.ds(0, CH), :], spm_sums.at[ia],
                              ss).wait()
        pltpu.make_async_copy(xb.at[pl.ds(CH, CH), :], spm_sums.at[ib2],
                              ss).wait()
        pltpu.make_async_copy(ones_buf, spm_cnts.at[ia], scn).wait()
        pltpu.make_async_copy(ones_buf, spm_cnts.at[ib2], scn).wait()

    pltpu.sync_copy(bat_hbm.at[pl.ds(base + 16, 16)], b0.at[pl.ds(0, 16)])
    cur0 = b0[pl.ds(0, 16)][0]

    def flush_run(sid, runc_l, m):
        first = runc_l == 0

        @pl.when(first)
        def _():
            for k in range(D // 16):
                ebuf[pl.ds(k * 16, 16)] = m[k]
            eid_buf[pl.ds(0, 16)] = jnp.full((16,), sid, jnp.int32)

        @pl.when(jnp.logical_not(first))
        def _():
            for k in range(D // 16):
                mflush[pl.ds(k * 16, 16)] = m[k]
            pltpu.sync_copy(mflush, maxh_hbm.at[sid, 0])

    def proc_rows(p, carry):
        xb, bb, ia, ib2, sx, sb, ss, scn = BUFS[p]

        def grp_body(gi, gc):
            cur = gc[0]
            runc_l = gc[1]
            m = list(gc[2:])
            gstart = gi * 16
            for r in range(16):
                sv = bb[pl.ds(16 + gstart + r, 16)][0]
                ch = sv != cur

                @pl.when(ch)
                def _(sid=cur, rl=runc_l, mm=tuple(m)):
                    flush_run(sid, rl, mm)

                newm = []
                for k in range(D // 16):
                    xk = xb[gstart + r, pl.ds(k * 16, 16)]
                    newm.append(jnp.where(ch, xk, jnp.maximum(m[k], xk)))
                m = newm
                runc_l = runc_l + ch.astype(jnp.int32)
                cur = sv
            return (cur, runc_l) + tuple(m)

        return lax.fori_loop(0, CHUNK // 16, grp_body, carry)

    issue_in(base, 0)
    carry = (cur0, jnp.int32(0)) + tuple(minf for _ in range(D // 16))
    wait_in(0)
    stage_idx(0)
    issue_scat(0)
    issue_in(base + CHUNK, 1)
    carry = proc_rows(0, carry)

    def pair_body(q, carry):
        cb = base + (2 * q + 1) * CHUNK
        wait_in(1)
        stage_idx(1)
        issue_scat(1)
        wait_scat(0)
        issue_in(cb + CHUNK, 0)
        carry = proc_rows(1, carry)

        wait_in(0)
        stage_idx(0)
        issue_scat(0)
        wait_scat(1)
        issue_in(cb + 2 * CHUNK, 1)
        carry = proc_rows(0, carry)
        return carry

    carry = lax.fori_loop(0, (NFULL - 2) // 2, pair_body, carry)

    cbl = base + rows_w - CHUNK
    wait_in(1)
    stage_idx(1)
    issue_scat(1)
    wait_scat(0)
    issue_in(cbl, 0)
    carry = proc_rows(1, carry)

    ovl = (NFULL + 1) * CHUNK - rows_w
    wait_in(0)
    for grp in range(CH // 16):
        bvs = b0[pl.ds(16 + grp * 16, 16)]
        pos = lanes + grp * 16
        i0a[pl.ds(grp * 16, 16)] = jnp.where(pos < ovl, jnp.int32(DUMMY), bvs)
        bvs2 = b0[pl.ds(16 + CH + grp * 16, 16)]
        pos2 = lanes + CH + grp * 16
        i0b[pl.ds(grp * 16, 16)] = jnp.where(pos2 < ovl, jnp.int32(DUMMY),
                                             bvs2)
    issue_scat(0)
    wait_scat(1)

    def row_body(j, rc):
        cur = rc[0]
        runc = rc[1]
        m = rc[2:]
        sv = b0[pl.ds(16 + j, 16)][0]
        changed = sv != cur

        @pl.when(changed)
        def _():
            flush_run(cur, runc, m)

        newm = []
        for k in range(D // 16):
            xk = x0[j, pl.ds(k * 16, 16)]
            newm.append(jnp.where(changed, xk, jnp.maximum(m[k], xk)))
        return (sv, runc + changed.astype(jnp.int32)) + tuple(newm)

    carry = lax.fori_loop(ovl, CHUNK, row_body, carry)
    wait_scat(0)

    cur = carry[0]
    runc = carry[1]
    m = carry[2:]
    for k in range(D // 16):
        ebuf[pl.ds(D + k * 16, 16)] = m[k]
    eid_buf[pl.ds(16, 16)] = jnp.full((16,), cur, jnp.int32)

    @pl.when(runc == 0)
    def _single_run():
        for k in range(D // 16):
            ebuf[pl.ds(k * 16, 16)] = m[k]
        eid_buf[pl.ds(0, 16)] = jnp.full((16,), cur, jnp.int32)

    pltpu.sync_copy(ebuf, emax_hbm.at[wid])
    pltpu.sync_copy(eid_buf, eid_hbm.at[wid])

    plsc.subcore_barrier()
    pltpu.sync_copy(spm_sums.at[pl.ds(rows0, BPS), :],
                    sums_hbm.at[c, pl.ds(rows0, BPS), :])
    pltpu.sync_copy(spm_cnts.at[pl.ds(rows0, BPS), :],
                    cnts_hbm.at[c, pl.ds(rows0, BPS), :])


def _tc_body(sums2, cnts2, maxh, emax, eid, w_ref, b_ref, z_ref, out_ref, mx):
    sums = sums2[0] + sums2[1]
    cnt = cnts2[0, :, 0:1] + cnts2[1, :, 0:1]
    mx[...] = jnp.where(cnt > 0.0, maxh[...], -jnp.inf)

    neg = jnp.full((1, D), -jnp.inf, jnp.float32)

    def _clear(i, _):
        sid = eid[i, 0]
        mx[pl.ds(sid, 1), :] = neg
        return 0
    lax.fori_loop(0, 2 * NW, _clear, 0)

    def _apply(i, _):
        sid = eid[i, 0]
        row = emax[pl.ds(i, 1), :]
        mx[pl.ds(sid, 1), :] = jnp.maximum(mx[pl.ds(sid, 1), :], row)
        return 0
    lax.fori_loop(0, 2 * NW, _apply, 0)

    mxv = mx[...]
    mxv = jnp.where(jnp.isfinite(mxv), mxv, 0.0)
    mean = sums / jnp.maximum(cnt, 1.0)
    z = jnp.concatenate([mean, mxv, sums], axis=1)
    z_ref[...] = z
    out_ref[...] = jnp.dot(z, w_ref[...],
                           preferred_element_type=jnp.float32) + b_ref[...]


def kernel(x, batch, W, b):
    batch_pad = jnp.concatenate([jnp.zeros((16,), jnp.int32), batch,
                                 jnp.zeros((BPAD,), jnp.int32)])

    mesh = plsc.VectorSubcoreMesh(core_axis_name="c", subcore_axis_name="s",
                                  num_cores=NC, num_subcores=NS)
    sc = pl.kernel(
        _sc_body,
        out_type=(
            jax.ShapeDtypeStruct((NC, B, D), jnp.float32),
            jax.ShapeDtypeStruct((NC, B, CNT_W), jnp.float32),
            jax.ShapeDtypeStruct((B, 1, D), jnp.float32),
            jax.ShapeDtypeStruct((NW, 2 * D), jnp.float32),
            jax.ShapeDtypeStruct((NW, 32), jnp.int32),
        ),
        mesh=mesh,
        scratch_types=[
            pltpu.VMEM((CHUNK, D), jnp.float32),
            pltpu.VMEM((BWIN,), jnp.int32),
            pltpu.VMEM((CH,), jnp.int32),
            pltpu.VMEM((CH,), jnp.int32),
            pltpu.VMEM((CHUNK, D), jnp.float32),
            pltpu.VMEM((BWIN,), jnp.int32),
            pltpu.VMEM((CH,), jnp.int32),
            pltpu.VMEM((CH,), jnp.int32),
            pltpu.VMEM((CH, CNT_W), jnp.float32),
            pltpu.VMEM((D,), jnp.float32),
            pltpu.VMEM((2 * D,), jnp.float32),
            pltpu.VMEM((32,), jnp.int32),
            pltpu.VMEM((BPS, D), jnp.float32),
            pltpu.VMEM((BPS, CNT_W), jnp.float32),
            pltpu.VMEM_SHARED((B + 8, D), jnp.float32),
            pltpu.VMEM_SHARED((B + 8, CNT_W), jnp.float32),
            pltpu.SemaphoreType.DMA,
            pltpu.SemaphoreType.DMA,
            pltpu.SemaphoreType.DMA,
            pltpu.SemaphoreType.DMA,
            pltpu.SemaphoreType.DMA,
            pltpu.SemaphoreType.DMA,
            pltpu.SemaphoreType.DMA,
            pltpu.SemaphoreType.DMA,
        ],
    )
    ones_arr = jnp.ones((CH, CNT_W), jnp.float32)
    sums2, cnts2, maxh, emax, eid = sc(x, batch_pad, ones_arr)

    z, logits = pl.pallas_call(
        _tc_body,
        out_shape=[
            jax.ShapeDtypeStruct((B, 3 * D), jnp.float32),
            jax.ShapeDtypeStruct((B, OUT), jnp.float32),
        ],
        in_specs=[
            pl.BlockSpec(memory_space=pltpu.VMEM),
            pl.BlockSpec(memory_space=pltpu.VMEM),
            pl.BlockSpec(memory_space=pltpu.VMEM),
            pl.BlockSpec(memory_space=pltpu.VMEM),
            pl.BlockSpec(memory_space=pltpu.SMEM),
            pl.BlockSpec(memory_space=pltpu.VMEM),
            pl.BlockSpec(memory_space=pltpu.VMEM),
        ],
        scratch_shapes=[pltpu.VMEM((B, D), jnp.float32)],
    )(sums2, cnts2, maxh.reshape(B, D), emax.reshape(2 * NW, D),
      eid.reshape(2 * NW, 16), W, b.reshape(1, OUT))
    return (z, logits)


# speedup vs baseline: 1.3211x; 1.2569x over previous
import jax
import jax.numpy as jnp
from jax import lax
from jax.experimental import pallas as pl
from jax.experimental.pallas import tpu as pltpu
from jax.experimental.pallas import tpu_sc as plsc

N = 100000
D = 128
B = 1024
OUT = 128

NC = 2
NS = 16
NW = NC * NS
RA = 3128
RB = 3120
NSPLIT = 20
CH = 128
NFULL = 24
BWIN = 160
BPAD = 32
CNT_W = 128
BPS = B // NS
DUMMY = B


def _sc_body(x_hbm, bat_hbm, ones_hbm, sums_hbm, cnts_hbm, maxh_hbm, emax_hbm,
             eid_hbm, x0, b0, i0, x1, b1, i1, ones_buf, mflush, ebuf, eid_buf,
             zrow, zcnt, spm_sums, spm_cnts,
             sx0, sb0, ss0, sc0, sx1, sb1, ss1, sc1):
    c = lax.axis_index("c")
    s = lax.axis_index("s")
    wid = c * NS + s

    zv = jnp.zeros((16,), jnp.float32)
    lanes = lax.iota(jnp.int32, 16)

    def _zfill(r, _):
        for k in range(D // 16):
            zrow[r, pl.ds(k * 16, 16)] = zv
            zcnt[r, pl.ds(k * 16, 16)] = zv
        return 0
    lax.fori_loop(0, BPS, _zfill, 0)

    pltpu.sync_copy(ones_hbm, ones_buf)

    rows0 = s * BPS
    pltpu.sync_copy(zrow, spm_sums.at[pl.ds(rows0, BPS), :])
    pltpu.sync_copy(zcnt, spm_cnts.at[pl.ds(rows0, BPS), :])
    plsc.subcore_barrier()

    base = jnp.where(wid < NSPLIT, wid * RA,
                     NSPLIT * RA + (wid - NSPLIT) * RB)
    rows_w = jnp.where(wid < NSPLIT, jnp.int32(RA), jnp.int32(RB))
    minf = jnp.full((16,), -jnp.inf, jnp.float32)

    BUFS = ((x0, b0, i0, sx0, sb0, ss0, sc0),
            (x1, b1, i1, sx1, sb1, ss1, sc1))

    def issue_in(cb, p):
        xb, bb, ib, sx, sb, ss, scn = BUFS[p]
        pltpu.async_copy(x_hbm.at[pl.ds(cb, CH), :], xb, sx)
        pltpu.async_copy(bat_hbm.at[pl.ds(cb, BWIN)], bb, sb)

    def wait_in(p):
        xb, bb, ib, sx, sb, ss, scn = BUFS[p]
        pltpu.make_async_copy(x_hbm.at[pl.ds(0, CH), :], xb, sx).wait()
        pltpu.make_async_copy(bat_hbm.at[pl.ds(0, BWIN)], bb, sb).wait()

    def stage_idx(p):
        xb, bb, ib, sx, sb, ss, scn = BUFS[p]
        for grp in range(CH // 16):
            ib[pl.ds(grp * 16, 16)] = bb[pl.ds(16 + grp * 16, 16)]

    def issue_scat(p):
        xb, bb, ib, sx, sb, ss, scn = BUFS[p]
        pltpu.async_copy(xb, spm_sums.at[ib], ss, add=True)
        pltpu.async_copy(ones_buf, spm_cnts.at[ib], scn, add=True)

    def wait_scat(p):
        xb, bb, ib, sx, sb, ss, scn = BUFS[p]
        pltpu.make_async_copy(xb, spm_sums.at[ib], ss).wait()
        pltpu.make_async_copy(ones_buf, spm_cnts.at[ib], scn).wait()

    pltpu.sync_copy(bat_hbm.at[pl.ds(base + 16, 16)], b0.at[pl.ds(0, 16)])
    cur0 = b0[pl.ds(0, 16)][0]

    def flush_run(sid, runc_l, m):
        first = runc_l == 0

        @pl.when(first)
        def _():
            for k in range(D // 16):
                ebuf[pl.ds(k * 16, 16)] = m[k]
            eid_buf[pl.ds(0, 16)] = jnp.full((16,), sid, jnp.int32)

        @pl.when(jnp.logical_not(first))
        def _():
            for k in range(D // 16):
                mflush[pl.ds(k * 16, 16)] = m[k]
            pltpu.sync_copy(mflush, maxh_hbm.at[sid, 0])

    lanes16 = lax.iota(jnp.int32, 16)

    def proc_rows(p, carry):
        xb, bb, ib, sx, sb, ss, scn = BUFS[p]

        def grp_body(gi, gc):
            gstart = gi * 16
            bv = bb[pl.ds(16 + gstart, 16)]
            pv = bb[pl.ds(15 + gstart, 16)]
            w = jnp.where(bv != pv, lanes16, jnp.int32(16))
            for sh in (8, 4, 2, 1):
                idx = (lanes16 + sh) & 15
                w = jnp.minimum(w, w.at[idx].get(mode="promise_in_bounds"))
            p_end = w[0]

            def fast_body(r, fm):
                out = []
                for k in range(D // 16):
                    out.append(jnp.maximum(fm[k],
                                           xb[gstart + r, pl.ds(k * 16, 16)]))
                return tuple(out)

            m = lax.fori_loop(0, p_end, fast_body, gc[2:])

            def slow_body(r, rc):
                cur = rc[0]
                runc_l = rc[1]
                sm = rc[2:]
                sv = bb[pl.ds(16 + gstart + r, 16)][0]
                ch = sv != cur

                @pl.when(ch)
                def _(sid=cur, rl=runc_l, mm=tuple(sm)):
                    flush_run(sid, rl, mm)

                newm = []
                for k in range(D // 16):
                    xk = xb[gstart + r, pl.ds(k * 16, 16)]
                    newm.append(jnp.where(ch, xk, jnp.maximum(sm[k], xk)))
                return (sv, runc_l + ch.astype(jnp.int32)) + tuple(newm)

            return lax.fori_loop(p_end, 16, slow_body, (gc[0], gc[1]) + m)

        return lax.fori_loop(0, CH // 16, grp_body, carry)

    issue_in(base, 0)
    carry = (cur0, jnp.int32(0)) + tuple(minf for _ in range(D // 16))
    wait_in(0)
    stage_idx(0)
    issue_scat(0)
    issue_in(base + CH, 1)
    carry = proc_rows(0, carry)

    def pair_body(q, carry):
        cb = base + (2 * q + 1) * CH
        wait_in(1)
        stage_idx(1)
        issue_scat(1)
        wait_scat(0)
        issue_in(cb + CH, 0)
        carry = proc_rows(1, carry)

        wait_in(0)
        stage_idx(0)
        issue_scat(0)
        wait_scat(1)
        issue_in(cb + 2 * CH, 1)
        carry = proc_rows(0, carry)
        return carry

    carry = lax.fori_loop(0, (NFULL - 2) // 2, pair_body, carry)

    cbl = base + rows_w - CH
    wait_in(1)
    stage_idx(1)
    issue_scat(1)
    wait_scat(0)
    issue_in(cbl, 0)
    carry = proc_rows(1, carry)

    ovl = (NFULL + 1) * CH - rows_w
    wait_in(0)
    for grp in range(CH // 16):
        bvs = b0[pl.ds(16 + grp * 16, 16)]
        pos = lanes + grp * 16
        i0[pl.ds(grp * 16, 16)] = jnp.where(pos < ovl, jnp.int32(DUMMY), bvs)
    issue_scat(0)
    wait_scat(1)

    def row_body(j, rc):
        cur = rc[0]
        runc = rc[1]
        m = rc[2:]
        sv = b0[pl.ds(16 + j, 16)][0]
        changed = sv != cur

        @pl.when(changed)
        def _():
            flush_run(cur, runc, m)

        newm = []
        for k in range(D // 16):
            xk = x0[j, pl.ds(k * 16, 16)]
            newm.append(jnp.where(changed, xk, jnp.maximum(m[k], xk)))
        return (sv, runc + changed.astype(jnp.int32)) + tuple(newm)

    carry = lax.fori_loop(ovl, CH, row_body, carry)
    wait_scat(0)

    cur = carry[0]
    runc = carry[1]
    m = carry[2:]
    for k in range(D // 16):
        ebuf[pl.ds(D + k * 16, 16)] = m[k]
    eid_buf[pl.ds(16, 16)] = jnp.full((16,), cur, jnp.int32)

    @pl.when(runc == 0)
    def _single_run():
        for k in range(D // 16):
            ebuf[pl.ds(k * 16, 16)] = m[k]
        eid_buf[pl.ds(0, 16)] = jnp.full((16,), cur, jnp.int32)

    pltpu.sync_copy(ebuf, emax_hbm.at[wid])
    pltpu.sync_copy(eid_buf, eid_hbm.at[wid])

    plsc.subcore_barrier()
    pltpu.sync_copy(spm_sums.at[pl.ds(rows0, BPS), :],
                    sums_hbm.at[c, pl.ds(rows0, BPS), :])
    pltpu.sync_copy(spm_cnts.at[pl.ds(rows0, BPS), :],
                    cnts_hbm.at[c, pl.ds(rows0, BPS), :])


def _tc_body(sums2, cnts2, maxh, emax, eid, w_ref, b_ref, z_ref, out_ref, mx):
    sums = sums2[0] + sums2[1]
    cnt = cnts2[0, :, 0:1] + cnts2[1, :, 0:1]
    mx[...] = jnp.where(cnt > 0.0, maxh[...], -jnp.inf)

    neg = jnp.full((1, D), -jnp.inf, jnp.float32)

    def _clear(i, _):
        sid = eid[i, 0]
        mx[pl.ds(sid, 1), :] = neg
        return 0
    lax.fori_loop(0, 2 * NW, _clear, 0)

    def _apply(i, _):
        sid = eid[i, 0]
        row = emax[pl.ds(i, 1), :]
        mx[pl.ds(sid, 1), :] = jnp.maximum(mx[pl.ds(sid, 1), :], row)
        return 0
    lax.fori_loop(0, 2 * NW, _apply, 0)

    mxv = mx[...]
    mxv = jnp.where(jnp.isfinite(mxv), mxv, 0.0)
    mean = sums / jnp.maximum(cnt, 1.0)
    z = jnp.concatenate([mean, mxv, sums], axis=1)
    z_ref[...] = z
    out_ref[...] = jnp.dot(z, w_ref[...],
                           preferred_element_type=jnp.float32) + b_ref[...]


def kernel(x, batch, W, b):
    batch_pad = jnp.concatenate([jnp.zeros((16,), jnp.int32), batch,
                                 jnp.zeros((BPAD,), jnp.int32)])

    mesh = plsc.VectorSubcoreMesh(core_axis_name="c", subcore_axis_name="s",
                                  num_cores=NC, num_subcores=NS)
    sc = pl.kernel(
        _sc_body,
        out_type=(
            jax.ShapeDtypeStruct((NC, B, D), jnp.float32),
            jax.ShapeDtypeStruct((NC, B, CNT_W), jnp.float32),
            jax.ShapeDtypeStruct((B, 1, D), jnp.float32),
            jax.ShapeDtypeStruct((NW, 2 * D), jnp.float32),
            jax.ShapeDtypeStruct((NW, 32), jnp.int32),
        ),
        mesh=mesh,
        scratch_types=[
            pltpu.VMEM((CH, D), jnp.float32),
            pltpu.VMEM((BWIN,), jnp.int32),
            pltpu.VMEM((CH,), jnp.int32),
            pltpu.VMEM((CH, D), jnp.float32),
            pltpu.VMEM((BWIN,), jnp.int32),
            pltpu.VMEM((CH,), jnp.int32),
            pltpu.VMEM((CH, CNT_W), jnp.float32),
            pltpu.VMEM((D,), jnp.float32),
            pltpu.VMEM((2 * D,), jnp.float32),
            pltpu.VMEM((32,), jnp.int32),
            pltpu.VMEM((BPS, D), jnp.float32),
            pltpu.VMEM((BPS, CNT_W), jnp.float32),
            pltpu.VMEM_SHARED((B + 8, D), jnp.float32),
            pltpu.VMEM_SHARED((B + 8, CNT_W), jnp.float32),
            pltpu.SemaphoreType.DMA,
            pltpu.SemaphoreType.DMA,
            pltpu.SemaphoreType.DMA,
            pltpu.SemaphoreType.DMA,
            pltpu.SemaphoreType.DMA,
            pltpu.SemaphoreType.DMA,
            pltpu.SemaphoreType.DMA,
            pltpu.SemaphoreType.DMA,
        ],
    )
    ones_arr = jnp.ones((CH, CNT_W), jnp.float32)
    sums2, cnts2, maxh, emax, eid = sc(x, batch_pad, ones_arr)

    z, logits = pl.pallas_call(
        _tc_body,
        out_shape=[
            jax.ShapeDtypeStruct((B, 3 * D), jnp.float32),
            jax.ShapeDtypeStruct((B, OUT), jnp.float32),
        ],
        in_specs=[
            pl.BlockSpec(memory_space=pltpu.VMEM),
            pl.BlockSpec(memory_space=pltpu.VMEM),
            pl.BlockSpec(memory_space=pltpu.VMEM),
            pl.BlockSpec(memory_space=pltpu.VMEM),
            pl.BlockSpec(memory_space=pltpu.SMEM),
            pl.BlockSpec(memory_space=pltpu.VMEM),
            pl.BlockSpec(memory_space=pltpu.VMEM),
        ],
        scratch_shapes=[pltpu.VMEM((B, D), jnp.float32)],
    )(sums2, cnts2, maxh.reshape(B, D), emax.reshape(2 * NW, D),
      eid.reshape(2 * NW, 16), W, b.reshape(1, OUT))
    return (z, logits)
